# baseline (speedup 1.0000x reference)
import functools

import jax
import jax.numpy as jnp
from jax import lax
from jax.experimental import pallas as pl
from jax.experimental.pallas import tpu as pltpu
from jax.experimental.pallas import tpu_sc as plsc

_CHUNK = 100
_WIN = 20
_NC = 2
_NS = 16
_BLK = 1000


def _sc_mesh():
    return plsc.VectorSubcoreMesh(core_axis_name="c", subcore_axis_name="s")


def _sc_degree(dst3d, ones1, zeros1, n_pad):
    nw, cpw, chunk = dst3d.shape

    @functools.partial(
        pl.kernel,
        mesh=_sc_mesh(),
        out_type=[
            jax.ShapeDtypeStruct((n_pad,), jnp.float32),
            jax.ShapeDtypeStruct((n_pad,), jnp.float32),
        ],
        scratch_types=[
            pltpu.VMEM((cpw, chunk), jnp.int32),
            pltpu.VMEM((chunk,), jnp.float32),
            pltpu.VMEM_SHARED((n_pad,), jnp.float32),
            pltpu.SemaphoreType.DMA,
        ],
    )
    def k(dst_hbm, ones_hbm, z_hbm, out0_hbm, out1_hbm, dstv, ones_v, acc,
          sem):
        c = lax.axis_index("c")
        s = lax.axis_index("s")
        w = c * _NS + s
        pltpu.sync_copy(dst_hbm.at[w], dstv)
        pltpu.sync_copy(ones_hbm, ones_v)

        @pl.when(s == 0)
        def _():
            pltpu.sync_copy(z_hbm, acc)

        plsc.subcore_barrier()

        def body(j, carry):
            pltpu.async_copy(ones_v, acc.at[dstv.at[j]], sem, add=True)
            return carry

        lax.fori_loop(0, cpw, body, 0)

        def drain(j, carry):
            pltpu.make_async_copy(ones_v, acc.at[dstv.at[j]], sem).wait()
            return carry

        lax.fori_loop(0, cpw, drain, 0)
        plsc.subcore_barrier()

        @pl.when(jnp.logical_and(s == 0, c == 0))
        def _():
            pltpu.sync_copy(acc, out0_hbm)

        @pl.when(jnp.logical_and(s == 0, c == 1))
        def _():
            pltpu.sync_copy(acc, out1_hbm)

    return k(dst3d, ones1, zeros1)


def _sc_scatter(g, src4d, dst4d, zeros, n_pad, d):
    nw, nwin, _, chunk = src4d.shape
    cpw = nwin * _WIN
    nbuf = 2

    @functools.partial(
        pl.kernel,
        mesh=_sc_mesh(),
        out_type=jax.ShapeDtypeStruct((_NC, n_pad, d), jnp.float32),
        scratch_types=[
            pltpu.VMEM((2, _WIN, chunk), jnp.int32),
            pltpu.VMEM((_WIN, chunk), jnp.int32),
            pltpu.VMEM((nbuf, chunk, d), jnp.float32),
            pltpu.VMEM_SHARED((n_pad, d), jnp.float32),
        ] + [pltpu.SemaphoreType.DMA] * (2 * nbuf + 1),
    )
    def k(g_hbm, src_hbm, dst_hbm, z_hbm, out_hbm, srcw, dstw, rows, acc,
          *sems):
        gsem = sems[:nbuf]
        ssem = sems[nbuf:2 * nbuf]
        isem = sems[2 * nbuf]
        c = lax.axis_index("c")
        s = lax.axis_index("s")
        w = c * _NS + s
        rpt = n_pad // _NS
        pltpu.sync_copy(src_hbm.at[w, 0], srcw.at[0])
        r0 = s * rpt
        pltpu.sync_copy(z_hbm.at[pl.ds(r0, rpt)], acc.at[pl.ds(r0, rpt)])
        plsc.subcore_barrier()

        look = 1
        for b in range(look):
            pltpu.async_copy(g_hbm.at[srcw.at[0].at[b]], rows.at[b], gsem[b])

        def body(win, carry):
            pltpu.sync_copy(dst_hbm.at[w, win], dstw)

            @pl.when(win + 1 < nwin)
            def _():
                pltpu.async_copy(src_hbm.at[w, win + 1],
                                 srcw.at[(win + 1) % 2], isem)

            for t in range(_WIN):
                b = t % nbuf
                b2 = (t + look) % nbuf
                j = win * _WIN + t

                pltpu.make_async_copy(g_hbm.at[srcw.at[win % 2].at[t]],
                                      rows.at[b], gsem[b]).wait()
                pltpu.async_copy(rows.at[b], acc.at[dstw.at[t]], ssem[b],
                                 add=True)

                @pl.when(j >= nbuf - look)
                def _():
                    pltpu.make_async_copy(rows.at[b2], acc.at[dstw.at[t]],
                                          ssem[b2]).wait()

                if t + look < _WIN:
                    pltpu.async_copy(g_hbm.at[srcw.at[win % 2].at[t + look]],
                                     rows.at[b2], gsem[b2])
                else:
                    @pl.when(win + 1 < nwin)
                    def _():
                        if t + look == _WIN:
                            pltpu.make_async_copy(src_hbm.at[w, win],
                                                  srcw.at[(win + 1) % 2],
                                                  isem).wait()
                        pltpu.async_copy(
                            g_hbm.at[srcw.at[(win + 1) % 2].at[t + look - _WIN]],
                            rows.at[b2], gsem[b2])
            return carry

        lax.fori_loop(0, nwin, body, 0)
        for tt in range(_WIN - (nbuf - look), _WIN):
            pltpu.make_async_copy(rows.at[tt % nbuf], acc.at[dstw.at[tt]],
                                  ssem[tt % nbuf]).wait()
        plsc.subcore_barrier()
        pltpu.sync_copy(acc.at[pl.ds(r0, rpt)], out_hbm.at[c, pl.ds(r0, rpt)])

    return k(g, src4d, dst4d, zeros)


def _tc_k0(deg0, deg1, x, W1, b1, Wg0):
    n, d = x.shape
    grid = n // _BLK

    def body(d0_ref, d1_ref, x_ref, W1_ref, b1_ref, Wg0_ref, dinv_ref, g0_ref):
        deg = d0_ref[...] + d1_ref[...] + 1.0
        dinv = lax.rsqrt(jnp.maximum(deg, 1.0))
        dinv_ref[...] = dinv
        h1 = jnp.dot(x_ref[...], W1_ref[...],
                     preferred_element_type=jnp.float32) + b1_ref[...]
        g0_ref[...] = jnp.dot(h1, Wg0_ref[...],
                              preferred_element_type=jnp.float32) * dinv

    return pl.pallas_call(
        body,
        grid=(grid,),
        in_specs=[
            pl.BlockSpec((_BLK, 1), lambda i: (i, 0)),
            pl.BlockSpec((_BLK, 1), lambda i: (i, 0)),
            pl.BlockSpec((_BLK, d), lambda i: (i, 0)),
            pl.BlockSpec(W1.shape, lambda i: (0, 0)),
            pl.BlockSpec((1, d), lambda i: (0, 0)),
            pl.BlockSpec(Wg0.shape, lambda i: (0, 0)),
        ],
        out_specs=[
            pl.BlockSpec((_BLK, 1), lambda i: (i, 0)),
            pl.BlockSpec((_BLK, d), lambda i: (i, 0)),
        ],
        out_shape=[
            jax.ShapeDtypeStruct((n, 1), jnp.float32),
            jax.ShapeDtypeStruct((n, d), jnp.float32),
        ],
    )(deg0, deg1, x, W1, b1.reshape(1, -1), Wg0)


def _tc_mid(p, gprev, dinv, bg, Wnext, jsum):
    n, d = gprev.shape
    grid = n // _BLK
    have_jsum = jsum is not None

    def body(p_ref, g_ref, dinv_ref, bg_ref, W_ref, *rest):
        if have_jsum:
            jsum_ref, gnext_ref, jout_ref = rest
        else:
            gnext_ref, jout_ref = rest
        dinv = dinv_ref[...]
        agg = (p_ref[0] + p_ref[1] + g_ref[...]) * dinv
        h = jnp.maximum(agg + bg_ref[...], 0.0)
        jout_ref[...] = (jsum_ref[...] + h) if have_jsum else h
        gnext_ref[...] = jnp.dot(h, W_ref[...],
                                 preferred_element_type=jnp.float32) * dinv

    in_specs = [
        pl.BlockSpec((_NC, _BLK, d), lambda i: (0, i, 0)),
        pl.BlockSpec((_BLK, d), lambda i: (i, 0)),
        pl.BlockSpec((_BLK, 1), lambda i: (i, 0)),
        pl.BlockSpec((1, d), lambda i: (0, 0)),
        pl.BlockSpec(Wnext.shape, lambda i: (0, 0)),
    ]
    args = [p, gprev, dinv, bg.reshape(1, -1), Wnext]
    if have_jsum:
        in_specs.append(pl.BlockSpec((_BLK, d), lambda i: (i, 0)))
        args.append(jsum)
    return pl.pallas_call(
        body,
        grid=(grid,),
        in_specs=in_specs,
        out_specs=[
            pl.BlockSpec((_BLK, d), lambda i: (i, 0)),
            pl.BlockSpec((_BLK, d), lambda i: (i, 0)),
        ],
        out_shape=[
            jax.ShapeDtypeStruct((n, d), jnp.float32),
            jax.ShapeDtypeStruct((n, d), jnp.float32),
        ],
    )(*args)


def _tc_k3(p, gprev, dinv, bg, jsum, Wnc, bnc):
    n, d = gprev.shape
    c = Wnc.shape[1]
    grid = n // _BLK

    def body(p_ref, g_ref, dinv_ref, bg_ref, jsum_ref, Wnc_ref, bnc_ref, out_ref):
        agg = (p_ref[0] + p_ref[1] + g_ref[...]) * dinv_ref[...]
        h = jnp.maximum(agg + bg_ref[...], 0.0)
        x5 = jsum_ref[...] + h
        out_ref[...] = jnp.dot(x5, Wnc_ref[...],
                               preferred_element_type=jnp.float32) + bnc_ref[...]

    return pl.pallas_call(
        body,
        grid=(grid,),
        in_specs=[
            pl.BlockSpec((_NC, _BLK, d), lambda i: (0, i, 0)),
            pl.BlockSpec((_BLK, d), lambda i: (i, 0)),
            pl.BlockSpec((_BLK, 1), lambda i: (i, 0)),
            pl.BlockSpec((1, d), lambda i: (0, 0)),
            pl.BlockSpec((_BLK, d), lambda i: (i, 0)),
            pl.BlockSpec(Wnc.shape, lambda i: (0, 0)),
            pl.BlockSpec((1, c), lambda i: (0, 0)),
        ],
        out_specs=pl.BlockSpec((_BLK, c), lambda i: (i, 0)),
        out_shape=jax.ShapeDtypeStruct((n, c), jnp.float32),
    )(p, gprev, dinv, bg.reshape(1, -1), jsum, Wnc, bnc.reshape(1, -1))


def kernel(x, edge_index, W1, b1, Wg0, bg0, Wg1, bg1, Wg2, bg2, Wnc, bnc):
    n, d = x.shape
    e = edge_index.shape[1]
    nw = _NC * _NS
    cpw = e // (_CHUNK * nw)
    n_pad = ((n + 8 * _NS - 1) // (8 * _NS)) * (8 * _NS)
    dst3d = edge_index[1].reshape(nw, cpw, _CHUNK)
    src4d = edge_index[0].reshape(nw, cpw // _WIN, _WIN, _CHUNK)
    dst4d = edge_index[1].reshape(nw, cpw // _WIN, _WIN, _CHUNK)
    zeros = jnp.zeros((n_pad, d), jnp.float32)
    zeros1 = jnp.zeros((n_pad,), jnp.float32)
    ones1 = jnp.ones((_CHUNK,), jnp.float32)

    deg0, deg1 = _sc_degree(dst3d, ones1, zeros1, n_pad)
    dinv, g0 = _tc_k0(deg0.reshape(-1, 1)[:n], deg1.reshape(-1, 1)[:n],
                      x, W1, b1, Wg0)
    p = _sc_scatter(g0, src4d, dst4d, zeros, n_pad, d)
    g1, j1 = _tc_mid(p, g0, dinv, bg0, Wg1, None)
    p = _sc_scatter(g1, src4d, dst4d, zeros, n_pad, d)
    g2, j2 = _tc_mid(p, g1, dinv, bg1, Wg2, j1)
    p = _sc_scatter(g2, src4d, dst4d, zeros, n_pad, d)
    return _tc_k3(p, g2, dinv, bg2, j2, Wnc, bnc)


# speedup vs baseline: 1.0439x; 1.0439x over previous
import functools

import jax
import jax.numpy as jnp
from jax import lax
from jax.experimental import pallas as pl
from jax.experimental.pallas import tpu as pltpu
from jax.experimental.pallas import tpu_sc as plsc

_CHUNK = 50
_WIN = 20
_NC = 2
_NS = 16
_BLK = 1000


def _sc_mesh():
    return plsc.VectorSubcoreMesh(core_axis_name="c", subcore_axis_name="s")


def _sc_degree(dst3d, ones1, zeros1, n_pad):
    nw, cpw, chunk = dst3d.shape

    @functools.partial(
        pl.kernel,
        mesh=_sc_mesh(),
        out_type=[
            jax.ShapeDtypeStruct((n_pad,), jnp.float32),
            jax.ShapeDtypeStruct((n_pad,), jnp.float32),
        ],
        scratch_types=[
            pltpu.VMEM((cpw, chunk), jnp.int32),
            pltpu.VMEM((chunk,), jnp.float32),
            pltpu.VMEM_SHARED((n_pad,), jnp.float32),
            pltpu.SemaphoreType.DMA,
        ],
    )
    def k(dst_hbm, ones_hbm, z_hbm, out0_hbm, out1_hbm, dstv, ones_v, acc,
          sem):
        c = lax.axis_index("c")
        s = lax.axis_index("s")
        w = c * _NS + s
        pltpu.sync_copy(dst_hbm.at[w], dstv)
        pltpu.sync_copy(ones_hbm, ones_v)

        @pl.when(s == 0)
        def _():
            pltpu.sync_copy(z_hbm, acc)

        plsc.subcore_barrier()

        def body(j, carry):
            pltpu.async_copy(ones_v, acc.at[dstv.at[j]], sem, add=True)
            return carry

        lax.fori_loop(0, cpw, body, 0)

        def drain(j, carry):
            pltpu.make_async_copy(ones_v, acc.at[dstv.at[j]], sem).wait()
            return carry

        lax.fori_loop(0, cpw, drain, 0)
        plsc.subcore_barrier()

        @pl.when(jnp.logical_and(s == 0, c == 0))
        def _():
            pltpu.sync_copy(acc, out0_hbm)

        @pl.when(jnp.logical_and(s == 0, c == 1))
        def _():
            pltpu.sync_copy(acc, out1_hbm)

    return k(dst3d, ones1, zeros1)


def _sc_scatter(g, src4d, dst4d, zeros, n_pad, d):
    nw, nwin, _, chunk = src4d.shape
    cpw = nwin * _WIN
    nbuf = 5

    @functools.partial(
        pl.kernel,
        mesh=_sc_mesh(),
        out_type=jax.ShapeDtypeStruct((_NC, n_pad, d), jnp.float32),
        scratch_types=[
            pltpu.VMEM((2, _WIN, chunk), jnp.int32),
            pltpu.VMEM((_WIN, chunk), jnp.int32),
            pltpu.VMEM((nbuf, chunk, d), jnp.float32),
            pltpu.VMEM_SHARED((n_pad, d), jnp.float32),
        ] + [pltpu.SemaphoreType.DMA] * (2 * nbuf + 1),
    )
    def k(g_hbm, src_hbm, dst_hbm, z_hbm, out_hbm, srcw, dstw, rows, acc,
          *sems):
        gsem = sems[:nbuf]
        ssem = sems[nbuf:2 * nbuf]
        isem = sems[2 * nbuf]
        c = lax.axis_index("c")
        s = lax.axis_index("s")
        w = c * _NS + s
        rpt = n_pad // _NS
        pltpu.sync_copy(src_hbm.at[w, 0], srcw.at[0])

        look = 2
        for b in range(look):
            pltpu.async_copy(g_hbm.at[srcw.at[0].at[b]], rows.at[b], gsem[b])

        r0 = s * rpt
        pltpu.sync_copy(z_hbm.at[pl.ds(r0, rpt)], acc.at[pl.ds(r0, rpt)])
        plsc.subcore_barrier()

        def body(win, carry):
            pltpu.sync_copy(dst_hbm.at[w, win], dstw)

            @pl.when(win + 1 < nwin)
            def _():
                pltpu.async_copy(src_hbm.at[w, win + 1],
                                 srcw.at[(win + 1) % 2], isem)

            for t in range(_WIN):
                b = t % nbuf
                b2 = (t + look) % nbuf
                j = win * _WIN + t

                pltpu.make_async_copy(g_hbm.at[srcw.at[win % 2].at[t]],
                                      rows.at[b], gsem[b]).wait()
                pltpu.async_copy(rows.at[b], acc.at[dstw.at[t]], ssem[b],
                                 add=True)

                @pl.when(j >= nbuf - look)
                def _():
                    pltpu.make_async_copy(rows.at[b2], acc.at[dstw.at[t]],
                                          ssem[b2]).wait()

                if t + look < _WIN:
                    pltpu.async_copy(g_hbm.at[srcw.at[win % 2].at[t + look]],
                                     rows.at[b2], gsem[b2])
                else:
                    @pl.when(win + 1 < nwin)
                    def _():
                        if t + look == _WIN:
                            pltpu.make_async_copy(src_hbm.at[w, win],
                                                  srcw.at[(win + 1) % 2],
                                                  isem).wait()
                        pltpu.async_copy(
                            g_hbm.at[srcw.at[(win + 1) % 2].at[t + look - _WIN]],
                            rows.at[b2], gsem[b2])
            return carry

        lax.fori_loop(0, nwin, body, 0)
        for tt in range(_WIN - (nbuf - look), _WIN):
            pltpu.make_async_copy(rows.at[tt % nbuf], acc.at[dstw.at[tt]],
                                  ssem[tt % nbuf]).wait()
        plsc.subcore_barrier()
        pltpu.sync_copy(acc.at[pl.ds(r0, rpt)], out_hbm.at[c, pl.ds(r0, rpt)])

    return k(g, src4d, dst4d, zeros)


def _tc_k0a(x, W1, b1, Wg0):
    n, d = x.shape
    grid = n // _BLK

    def body(x_ref, W1_ref, b1_ref, Wg0_ref, ht_ref):
        h1 = jnp.dot(x_ref[...], W1_ref[...],
                     preferred_element_type=jnp.float32) + b1_ref[...]
        ht_ref[...] = jnp.dot(h1, Wg0_ref[...],
                              preferred_element_type=jnp.float32)

    return pl.pallas_call(
        body,
        grid=(grid,),
        in_specs=[
            pl.BlockSpec((_BLK, d), lambda i: (i, 0)),
            pl.BlockSpec(W1.shape, lambda i: (0, 0)),
            pl.BlockSpec((1, d), lambda i: (0, 0)),
            pl.BlockSpec(Wg0.shape, lambda i: (0, 0)),
        ],
        out_specs=pl.BlockSpec((_BLK, d), lambda i: (i, 0)),
        out_shape=jax.ShapeDtypeStruct((n, d), jnp.float32),
    )(x, W1, b1.reshape(1, -1), Wg0)


def _tc_k0b(deg0, deg1, ht0):
    n, d = ht0.shape
    grid = n // _BLK

    def body(d0_ref, d1_ref, ht_ref, dinv_ref, g0_ref):
        deg = d0_ref[...] + d1_ref[...] + 1.0
        dinv = lax.rsqrt(jnp.maximum(deg, 1.0))
        dinv_ref[...] = dinv
        g0_ref[...] = ht_ref[...] * dinv

    return pl.pallas_call(
        body,
        grid=(grid,),
        in_specs=[
            pl.BlockSpec((_BLK, 1), lambda i: (i, 0)),
            pl.BlockSpec((_BLK, 1), lambda i: (i, 0)),
            pl.BlockSpec((_BLK, d), lambda i: (i, 0)),
        ],
        out_specs=[
            pl.BlockSpec((_BLK, 1), lambda i: (i, 0)),
            pl.BlockSpec((_BLK, d), lambda i: (i, 0)),
        ],
        out_shape=[
            jax.ShapeDtypeStruct((n, 1), jnp.float32),
            jax.ShapeDtypeStruct((n, d), jnp.float32),
        ],
    )(deg0, deg1, ht0)


def _tc_mid(p, gprev, dinv, bg, Wnext, jsum):
    n, d = gprev.shape
    grid = n // _BLK
    have_jsum = jsum is not None

    def body(p_ref, g_ref, dinv_ref, bg_ref, W_ref, *rest):
        if have_jsum:
            jsum_ref, gnext_ref, jout_ref = rest
        else:
            gnext_ref, jout_ref = rest
        dinv = dinv_ref[...]
        agg = (p_ref[0] + p_ref[1] + g_ref[...]) * dinv
        h = jnp.maximum(agg + bg_ref[...], 0.0)
        jout_ref[...] = (jsum_ref[...] + h) if have_jsum else h
        gnext_ref[...] = jnp.dot(h, W_ref[...],
                                 preferred_element_type=jnp.float32) * dinv

    in_specs = [
        pl.BlockSpec((_NC, _BLK, d), lambda i: (0, i, 0)),
        pl.BlockSpec((_BLK, d), lambda i: (i, 0)),
        pl.BlockSpec((_BLK, 1), lambda i: (i, 0)),
        pl.BlockSpec((1, d), lambda i: (0, 0)),
        pl.BlockSpec(Wnext.shape, lambda i: (0, 0)),
    ]
    args = [p, gprev, dinv, bg.reshape(1, -1), Wnext]
    if have_jsum:
        in_specs.append(pl.BlockSpec((_BLK, d), lambda i: (i, 0)))
        args.append(jsum)
    return pl.pallas_call(
        body,
        grid=(grid,),
        in_specs=in_specs,
        out_specs=[
            pl.BlockSpec((_BLK, d), lambda i: (i, 0)),
            pl.BlockSpec((_BLK, d), lambda i: (i, 0)),
        ],
        out_shape=[
            jax.ShapeDtypeStruct((n, d), jnp.float32),
            jax.ShapeDtypeStruct((n, d), jnp.float32),
        ],
    )(*args)


def _tc_k3(p, gprev, dinv, bg, jsum, Wnc, bnc):
    n, d = gprev.shape
    c = Wnc.shape[1]
    grid = n // _BLK

    def body(p_ref, g_ref, dinv_ref, bg_ref, jsum_ref, Wnc_ref, bnc_ref, out_ref):
        agg = (p_ref[0] + p_ref[1] + g_ref[...]) * dinv_ref[...]
        h = jnp.maximum(agg + bg_ref[...], 0.0)
        x5 = jsum_ref[...] + h
        out_ref[...] = jnp.dot(x5, Wnc_ref[...],
                               preferred_element_type=jnp.float32) + bnc_ref[...]

    return pl.pallas_call(
        body,
        grid=(grid,),
        in_specs=[
            pl.BlockSpec((_NC, _BLK, d), lambda i: (0, i, 0)),
            pl.BlockSpec((_BLK, d), lambda i: (i, 0)),
            pl.BlockSpec((_BLK, 1), lambda i: (i, 0)),
            pl.BlockSpec((1, d), lambda i: (0, 0)),
            pl.BlockSpec((_BLK, d), lambda i: (i, 0)),
            pl.BlockSpec(Wnc.shape, lambda i: (0, 0)),
            pl.BlockSpec((1, c), lambda i: (0, 0)),
        ],
        out_specs=pl.BlockSpec((_BLK, c), lambda i: (i, 0)),
        out_shape=jax.ShapeDtypeStruct((n, c), jnp.float32),
    )(p, gprev, dinv, bg.reshape(1, -1), jsum, Wnc, bnc.reshape(1, -1))


def kernel(x, edge_index, W1, b1, Wg0, bg0, Wg1, bg1, Wg2, bg2, Wnc, bnc):
    n, d = x.shape
    e = edge_index.shape[1]
    nw = _NC * _NS
    cpw = e // (_CHUNK * nw)
    n_pad = ((n + 8 * _NS - 1) // (8 * _NS)) * (8 * _NS)
    dst3d = edge_index[1].reshape(nw, cpw, _CHUNK)
    src4d = edge_index[0].reshape(nw, cpw // _WIN, _WIN, _CHUNK)
    dst4d = edge_index[1].reshape(nw, cpw // _WIN, _WIN, _CHUNK)
    zeros = jnp.zeros((n_pad, d), jnp.float32)
    zeros1 = jnp.zeros((n_pad,), jnp.float32)
    ones1 = jnp.ones((_CHUNK,), jnp.float32)

    deg0, deg1 = _sc_degree(dst3d, ones1, zeros1, n_pad)
    ht0 = _tc_k0a(x, W1, b1, Wg0)
    dinv, g0 = _tc_k0b(deg0.reshape(-1, 1)[:n], deg1.reshape(-1, 1)[:n], ht0)
    p = _sc_scatter(g0, src4d, dst4d, zeros, n_pad, d)
    g1, j1 = _tc_mid(p, g0, dinv, bg0, Wg1, None)
    p = _sc_scatter(g1, src4d, dst4d, zeros, n_pad, d)
    g2, j2 = _tc_mid(p, g1, dinv, bg1, Wg2, j1)
    p = _sc_scatter(g2, src4d, dst4d, zeros, n_pad, d)
    return _tc_k3(p, g2, dinv, bg2, j2, Wnc, bnc)


# speedup vs baseline: 1.0589x; 1.0143x over previous
import functools

import jax
import jax.numpy as jnp
from jax import lax
from jax.experimental import pallas as pl
from jax.experimental.pallas import tpu as pltpu
from jax.experimental.pallas import tpu_sc as plsc

_CHUNK = 50
_WIN = 20
_NC = 2
_NS = 16
_BLK = 1000


def _sc_mesh():
    return plsc.VectorSubcoreMesh(core_axis_name="c", subcore_axis_name="s")


def _sc_degree(dst3d, ones1, zeros1, n_pad):
    nw, cpw, chunk = dst3d.shape

    @functools.partial(
        pl.kernel,
        mesh=_sc_mesh(),
        out_type=[
            jax.ShapeDtypeStruct((n_pad,), jnp.float32),
            jax.ShapeDtypeStruct((n_pad,), jnp.float32),
        ],
        scratch_types=[
            pltpu.VMEM((cpw, chunk), jnp.int32),
            pltpu.VMEM((chunk,), jnp.float32),
            pltpu.VMEM_SHARED((n_pad,), jnp.float32),
            pltpu.SemaphoreType.DMA,
        ],
    )
    def k(dst_hbm, ones_hbm, z_hbm, out0_hbm, out1_hbm, dstv, ones_v, acc,
          sem):
        c = lax.axis_index("c")
        s = lax.axis_index("s")
        w = c * _NS + s
        pltpu.sync_copy(dst_hbm.at[w], dstv)
        pltpu.sync_copy(ones_hbm, ones_v)

        @pl.when(s == 0)
        def _():
            pltpu.sync_copy(z_hbm, acc)

        plsc.subcore_barrier()

        def body(j, carry):
            pltpu.async_copy(ones_v, acc.at[dstv.at[j]], sem, add=True)
            return carry

        lax.fori_loop(0, cpw, body, 0)

        def drain(j, carry):
            pltpu.make_async_copy(ones_v, acc.at[dstv.at[j]], sem).wait()
            return carry

        lax.fori_loop(0, cpw, drain, 0)
        plsc.subcore_barrier()

        @pl.when(jnp.logical_and(s == 0, c == 0))
        def _():
            pltpu.sync_copy(acc, out0_hbm)

        @pl.when(jnp.logical_and(s == 0, c == 1))
        def _():
            pltpu.sync_copy(acc, out1_hbm)

    return k(dst3d, ones1, zeros1)


def _sc_scatter(g, src4d, dst4d, zeros, n_pad, d):
    nw, nwin, _, chunk = src4d.shape
    cpw = nwin * _WIN
    nbuf = 5

    @functools.partial(
        pl.kernel,
        mesh=_sc_mesh(),
        out_type=jax.ShapeDtypeStruct((_NC, n_pad, d), jnp.float32),
        scratch_types=[
            pltpu.VMEM((2, _WIN, chunk), jnp.int32),
            pltpu.VMEM((_WIN, chunk), jnp.int32),
            pltpu.VMEM((nbuf, chunk, d), jnp.float32),
            pltpu.VMEM_SHARED((n_pad, d), jnp.float32),
        ] + [pltpu.SemaphoreType.DMA] * (2 * nbuf + 1),
    )
    def k(g_hbm, src_hbm, dst_hbm, z_hbm, out_hbm, srcw, dstw, rows, acc,
          *sems):
        gsem = sems[:nbuf]
        ssem = sems[nbuf:2 * nbuf]
        isem = sems[2 * nbuf]
        c = lax.axis_index("c")
        s = lax.axis_index("s")
        w = c * _NS + s
        rpt = n_pad // _NS
        pltpu.sync_copy(src_hbm.at[w, 0], srcw.at[0])

        look = 2
        for b in range(look):
            pltpu.async_copy(g_hbm.at[srcw.at[0].at[b]], rows.at[b], gsem[b])

        r0 = s * rpt
        pltpu.sync_copy(z_hbm.at[pl.ds(r0, rpt)], acc.at[pl.ds(r0, rpt)])
        plsc.subcore_barrier()

        def body(win, carry):
            pltpu.sync_copy(dst_hbm.at[w, win], dstw)

            @pl.when(win + 1 < nwin)
            def _():
                pltpu.async_copy(src_hbm.at[w, win + 1],
                                 srcw.at[(win + 1) % 2], isem)

            for t in range(_WIN):
                b = t % nbuf
                b2 = (t + look) % nbuf
                j = win * _WIN + t

                pltpu.make_async_copy(g_hbm.at[srcw.at[win % 2].at[t]],
                                      rows.at[b], gsem[b]).wait()
                pltpu.async_copy(rows.at[b], acc.at[dstw.at[t]], ssem[b],
                                 add=True)

                @pl.when(j >= nbuf - look)
                def _():
                    pltpu.make_async_copy(rows.at[b2], acc.at[dstw.at[t]],
                                          ssem[b2]).wait()

                if t + look < _WIN:
                    pltpu.async_copy(g_hbm.at[srcw.at[win % 2].at[t + look]],
                                     rows.at[b2], gsem[b2])
                else:
                    @pl.when(win + 1 < nwin)
                    def _():
                        if t + look == _WIN:
                            pltpu.make_async_copy(src_hbm.at[w, win],
                                                  srcw.at[(win + 1) % 2],
                                                  isem).wait()
                        pltpu.async_copy(
                            g_hbm.at[srcw.at[(win + 1) % 2].at[t + look - _WIN]],
                            rows.at[b2], gsem[b2])
            return carry

        lax.fori_loop(0, nwin, body, 0)
        for tt in range(_WIN - (nbuf - look), _WIN):
            pltpu.make_async_copy(rows.at[tt % nbuf], acc.at[dstw.at[tt]],
                                  ssem[tt % nbuf]).wait()
        plsc.subcore_barrier()
        pltpu.sync_copy(acc.at[pl.ds(r0, rpt)], out_hbm.at[c, pl.ds(r0, rpt)])

    return k(g, src4d, dst4d, zeros)


def _tc_k0(deg0, deg1, x, W1, b1, Wg0):
    n, d = x.shape
    grid = n // _BLK

    def body(d0_ref, d1_ref, x_ref, W1_ref, b1_ref, Wg0_ref, dinv_ref, g0_ref):
        deg = d0_ref[...] + d1_ref[...] + 1.0
        dinv = lax.rsqrt(jnp.maximum(deg, 1.0))
        dinv_ref[...] = dinv
        h1 = jnp.dot(x_ref[...], W1_ref[...],
                     preferred_element_type=jnp.float32) + b1_ref[...]
        g0_ref[...] = jnp.dot(h1, Wg0_ref[...],
                              preferred_element_type=jnp.float32) * dinv

    return pl.pallas_call(
        body,
        grid=(grid,),
        in_specs=[
            pl.BlockSpec((_BLK, 1), lambda i: (i, 0)),
            pl.BlockSpec((_BLK, 1), lambda i: (i, 0)),
            pl.BlockSpec((_BLK, d), lambda i: (i, 0)),
            pl.BlockSpec(W1.shape, lambda i: (0, 0)),
            pl.BlockSpec((1, d), lambda i: (0, 0)),
            pl.BlockSpec(Wg0.shape, lambda i: (0, 0)),
        ],
        out_specs=[
            pl.BlockSpec((_BLK, 1), lambda i: (i, 0)),
            pl.BlockSpec((_BLK, d), lambda i: (i, 0)),
        ],
        out_shape=[
            jax.ShapeDtypeStruct((n, 1), jnp.float32),
            jax.ShapeDtypeStruct((n, d), jnp.float32),
        ],
    )(deg0, deg1, x, W1, b1.reshape(1, -1), Wg0)


def _tc_mid(p, gprev, dinv, bg, Wnext, jsum):
    n, d = gprev.shape
    grid = n // _BLK
    have_jsum = jsum is not None

    def body(p_ref, g_ref, dinv_ref, bg_ref, W_ref, *rest):
        if have_jsum:
            jsum_ref, gnext_ref, jout_ref = rest
        else:
            gnext_ref, jout_ref = rest
        dinv = dinv_ref[...]
        agg = (p_ref[0] + p_ref[1] + g_ref[...]) * dinv
        h = jnp.maximum(agg + bg_ref[...], 0.0)
        jout_ref[...] = (jsum_ref[...] + h) if have_jsum else h
        gnext_ref[...] = jnp.dot(h, W_ref[...],
                                 preferred_element_type=jnp.float32) * dinv

    in_specs = [
        pl.BlockSpec((_NC, _BLK, d), lambda i: (0, i, 0)),
        pl.BlockSpec((_BLK, d), lambda i: (i, 0)),
        pl.BlockSpec((_BLK, 1), lambda i: (i, 0)),
        pl.BlockSpec((1, d), lambda i: (0, 0)),
        pl.BlockSpec(Wnext.shape, lambda i: (0, 0)),
    ]
    args = [p, gprev, dinv, bg.reshape(1, -1), Wnext]
    if have_jsum:
        in_specs.append(pl.BlockSpec((_BLK, d), lambda i: (i, 0)))
        args.append(jsum)
    return pl.pallas_call(
        body,
        grid=(grid,),
        in_specs=in_specs,
        out_specs=[
            pl.BlockSpec((_BLK, d), lambda i: (i, 0)),
            pl.BlockSpec((_BLK, d), lambda i: (i, 0)),
        ],
        out_shape=[
            jax.ShapeDtypeStruct((n, d), jnp.float32),
            jax.ShapeDtypeStruct((n, d), jnp.float32),
        ],
    )(*args)


def _tc_k3(p, gprev, dinv, bg, jsum, Wnc, bnc):
    n, d = gprev.shape
    c = Wnc.shape[1]
    grid = n // _BLK

    def body(p_ref, g_ref, dinv_ref, bg_ref, jsum_ref, Wnc_ref, bnc_ref, out_ref):
        agg = (p_ref[0] + p_ref[1] + g_ref[...]) * dinv_ref[...]
        h = jnp.maximum(agg + bg_ref[...], 0.0)
        x5 = jsum_ref[...] + h
        out_ref[...] = jnp.dot(x5, Wnc_ref[...],
                               preferred_element_type=jnp.float32) + bnc_ref[...]

    return pl.pallas_call(
        body,
        grid=(grid,),
        in_specs=[
            pl.BlockSpec((_NC, _BLK, d), lambda i: (0, i, 0)),
            pl.BlockSpec((_BLK, d), lambda i: (i, 0)),
            pl.BlockSpec((_BLK, 1), lambda i: (i, 0)),
            pl.BlockSpec((1, d), lambda i: (0, 0)),
            pl.BlockSpec((_BLK, d), lambda i: (i, 0)),
            pl.BlockSpec(Wnc.shape, lambda i: (0, 0)),
            pl.BlockSpec((1, c), lambda i: (0, 0)),
        ],
        out_specs=pl.BlockSpec((_BLK, c), lambda i: (i, 0)),
        out_shape=jax.ShapeDtypeStruct((n, c), jnp.float32),
    )(p, gprev, dinv, bg.reshape(1, -1), jsum, Wnc, bnc.reshape(1, -1))


def kernel(x, edge_index, W1, b1, Wg0, bg0, Wg1, bg1, Wg2, bg2, Wnc, bnc):
    n, d = x.shape
    e = edge_index.shape[1]
    nw = _NC * _NS
    cpw = e // (_CHUNK * nw)
    n_pad = ((n + 8 * _NS - 1) // (8 * _NS)) * (8 * _NS)
    dst3d = edge_index[1].reshape(nw, cpw, _CHUNK)
    src4d = edge_index[0].reshape(nw, cpw // _WIN, _WIN, _CHUNK)
    dst4d = edge_index[1].reshape(nw, cpw // _WIN, _WIN, _CHUNK)
    zeros = jnp.zeros((n_pad, d), jnp.float32)
    zeros1 = jnp.zeros((n_pad,), jnp.float32)
    ones1 = jnp.ones((_CHUNK,), jnp.float32)

    deg0, deg1 = _sc_degree(dst3d, ones1, zeros1, n_pad)
    dinv, g0 = _tc_k0(deg0.reshape(-1, 1)[:n], deg1.reshape(-1, 1)[:n],
                      x, W1, b1, Wg0)
    p = _sc_scatter(g0, src4d, dst4d, zeros, n_pad, d)
    g1, j1 = _tc_mid(p, g0, dinv, bg0, Wg1, None)
    p = _sc_scatter(g1, src4d, dst4d, zeros, n_pad, d)
    g2, j2 = _tc_mid(p, g1, dinv, bg1, Wg2, j1)
    p = _sc_scatter(g2, src4d, dst4d, zeros, n_pad, d)
    return _tc_k3(p, g2, dinv, bg2, j2, Wnc, bnc)
